# baseline (speedup 1.0000x reference)
import functools

import jax
import jax.numpy as jnp
from jax import lax
from jax.experimental import pallas as pl
from jax.experimental.pallas import tpu as pltpu
from jax.experimental.pallas import tpu_sc as plsc

VOCAB = 1000000
EMBED = 64
MAXSEQ = 200
BATCH = 4096
SEQ = 200

TOTAL_ROWS = BATCH * SEQ
LANES = 16
VPR = EMBED // LANES

_INFO = plsc.get_sparse_core_info()
NC = _INFO.num_cores
NS = _INFO.num_subcores
NW = NC * NS
ROWS_PER_W = TOTAL_ROWS // NW
CHUNK = 128
NCHUNK = ROWS_PER_W // CHUNK


def _rsqrt(x):
    i = lax.bitcast_convert_type(x, jnp.int32)
    i = jnp.int32(0x5F3759DF) - lax.shift_right_logical(i, 1)
    y = lax.bitcast_convert_type(i, jnp.float32)
    for _ in range(3):
        y = y * (jnp.float32(1.5) - jnp.float32(0.5) * x * y * y)
    return y


def _make_sc_call():
    mesh = plsc.VectorSubcoreMesh(core_axis_name="c", subcore_axis_name="s")

    @functools.partial(
        pl.kernel,
        mesh=mesh,
        compiler_params=pltpu.CompilerParams(
            needs_layout_passes=False, use_tc_tiling_on_sc=False),
        out_type=jax.ShapeDtypeStruct((TOTAL_ROWS, EMBED), jnp.float32),
        scratch_types=[
            pltpu.VMEM((CHUNK,), jnp.int32),
            pltpu.VMEM((CHUNK, EMBED), jnp.float32),
            pltpu.VMEM((CHUNK, EMBED), jnp.float32),
            pltpu.VMEM((MAXSEQ, EMBED), jnp.float32),
            pltpu.VMEM((EMBED,), jnp.float32),
            pltpu.VMEM((EMBED,), jnp.float32),
            pltpu.SemaphoreType.DMA,
        ],
    )
    def sc_embed(ids_hbm, table_hbm, pos_hbm, gamma_hbm, beta_hbm, out_hbm,
                 idx_v, rows_v, out_stage, pos_v, gamma_v, beta_v, gsem):
        wid = lax.axis_index("s") * NC + lax.axis_index("c")
        wstart = wid * ROWS_PER_W

        pltpu.sync_copy(pos_hbm, pos_v)
        pltpu.sync_copy(gamma_hbm, gamma_v)
        pltpu.sync_copy(beta_hbm, beta_v)

        inv_n = jnp.float32(1.0 / EMBED)
        eps = jnp.float32(1e-5)

        def row_body(r, base):
            p = lax.rem(base + r, MAXSEQ)
            xs = []
            for k in range(VPR):
                t = rows_v[r, pl.ds(k * LANES, LANES)]
                q = pos_v[p, pl.ds(k * LANES, LANES)]
                xs.append(t + q)
            s = (xs[0] + xs[1]) + (xs[2] + xs[3])
            ssq = (xs[0] * xs[0] + xs[1] * xs[1]) + (xs[2] * xs[2] + xs[3] * xs[3])
            mean = jnp.sum(s) * inv_n
            var = jnp.sum(ssq) * inv_n - mean * mean
            rstd = _rsqrt(var + eps)
            mean_v = jnp.broadcast_to(mean, (LANES,))
            rstd_v = jnp.broadcast_to(rstd, (LANES,))
            for k in range(VPR):
                g = gamma_v[pl.ds(k * LANES, LANES)]
                b = beta_v[pl.ds(k * LANES, LANES)]
                xh = (xs[k] - mean_v) * rstd_v
                out_stage[r, pl.ds(k * LANES, LANES)] = xh * g + b
            return base

        def chunk_body(c, _):
            base = wstart + c * CHUNK
            pltpu.sync_copy(ids_hbm.at[pl.ds(base, CHUNK)], idx_v)
            pltpu.async_copy(table_hbm.at[idx_v], rows_v, gsem).wait()
            lax.fori_loop(0, CHUNK, row_body, base)
            pltpu.sync_copy(out_stage, out_hbm.at[pl.ds(base, CHUNK)])
            return 0

        lax.fori_loop(0, NCHUNK, chunk_body, 0)

    return sc_embed


_sc_embed = _make_sc_call()


@jax.jit
def _run(ids_flat, token_table, pos_table, gamma, beta):
    return _sc_embed(ids_flat, token_table, pos_table, gamma, beta)


def kernel(input_ids, token_table, pos_table, gamma, beta):
    ids_flat = input_ids.reshape(TOTAL_ROWS).astype(jnp.int32)
    out = _run(ids_flat, token_table, pos_table, gamma, beta)
    return out.reshape(BATCH, SEQ, EMBED)


# speedup vs baseline: 2.2497x; 2.2497x over previous
import functools

import jax
import jax.numpy as jnp
from jax import lax
from jax.experimental import pallas as pl
from jax.experimental.pallas import tpu as pltpu
from jax.experimental.pallas import tpu_sc as plsc

VOCAB = 1000000
EMBED = 64
MAXSEQ = 200
BATCH = 4096
SEQ = 200

TOTAL_ROWS = BATCH * SEQ
LANES = 16
VPR = EMBED // LANES

_INFO = plsc.get_sparse_core_info()
NC = _INFO.num_cores
NS = _INFO.num_subcores
NW = NC * NS
ROWS_PER_W = TOTAL_ROWS // NW
CHUNK = 128
NCHUNK = ROWS_PER_W // CHUNK


def _rsqrt(x):
    i = lax.bitcast_convert_type(x, jnp.int32)
    i = jnp.int32(0x5F3759DF) - lax.shift_right_logical(i, 1)
    y = lax.bitcast_convert_type(i, jnp.float32)
    for _ in range(3):
        y = y * (jnp.float32(1.5) - jnp.float32(0.5) * x * y * y)
    return y


def _make_sc_call():
    mesh = plsc.VectorSubcoreMesh(core_axis_name="c", subcore_axis_name="s")

    @functools.partial(
        pl.kernel,
        mesh=mesh,
        compiler_params=pltpu.CompilerParams(
            needs_layout_passes=False, use_tc_tiling_on_sc=False),
        out_type=jax.ShapeDtypeStruct((TOTAL_ROWS, EMBED), jnp.float32),
        scratch_types=[
            pltpu.VMEM((NCHUNK, CHUNK), jnp.int32),
            pltpu.VMEM((2, CHUNK, EMBED), jnp.float32),
            pltpu.VMEM((2, CHUNK, EMBED), jnp.float32),
            pltpu.VMEM((MAXSEQ, EMBED), jnp.float32),
            pltpu.VMEM((EMBED,), jnp.float32),
            pltpu.VMEM((EMBED,), jnp.float32),
            pltpu.SemaphoreType.DMA,
            pltpu.SemaphoreType.DMA,
            pltpu.SemaphoreType.DMA,
            pltpu.SemaphoreType.DMA,
        ],
    )
    def sc_embed(ids_hbm, table_hbm, pos_hbm, gamma_hbm, beta_hbm, out_hbm,
                 idx_all, rows2, ost2, pos_v, gamma_v, beta_v,
                 gsem0, gsem1, osem0, osem1):
        wid = lax.axis_index("s") * NC + lax.axis_index("c")
        wstart = wid * ROWS_PER_W
        gsems = (gsem0, gsem1)
        osems = (osem0, osem1)

        pltpu.sync_copy(ids_hbm.at[wid], idx_all)
        pltpu.sync_copy(pos_hbm, pos_v)
        pltpu.sync_copy(gamma_hbm, gamma_v)
        pltpu.sync_copy(beta_hbm, beta_v)

        inv_n = jnp.float32(1.0 / EMBED)
        eps = jnp.float32(1e-5)

        def fire_gather(c, b):
            pltpu.async_copy(table_hbm.at[idx_all.at[c]], rows2.at[b],
                             gsems[b])

        def wait_gather(c, b):
            pltpu.make_async_copy(table_hbm.at[idx_all.at[c]], rows2.at[b],
                                  gsems[b]).wait()

        def fire_out(base, b):
            pltpu.async_copy(ost2.at[b], out_hbm.at[pl.ds(base, CHUNK)],
                             osems[b])

        def wait_out(base, b):
            pltpu.make_async_copy(ost2.at[b], out_hbm.at[pl.ds(base, CHUNK)],
                                  osems[b]).wait()

        def process(c, b):
            base = wstart + c * CHUNK
            wait_gather(c, b)

            @pl.when(c >= 2)
            def _():
                wait_out(base, b)

            @plsc.parallel_loop(0, CHUNK, unroll=4)
            def _row(r):
                p = lax.rem(base + r, MAXSEQ)
                xs = []
                for k in range(VPR):
                    t = rows2[b, r, pl.ds(k * LANES, LANES)]
                    q = pos_v[p, pl.ds(k * LANES, LANES)]
                    xs.append(t + q)
                s = (xs[0] + xs[1]) + (xs[2] + xs[3])
                ssq = (xs[0] * xs[0] + xs[1] * xs[1]) + \
                      (xs[2] * xs[2] + xs[3] * xs[3])
                mean = jnp.sum(s) * inv_n
                var = jnp.sum(ssq) * inv_n - mean * mean
                rstd = _rsqrt(var + eps)
                scale = jnp.broadcast_to(rstd, (LANES,))
                mean_v = jnp.broadcast_to(mean, (LANES,))
                for k in range(VPR):
                    g = gamma_v[pl.ds(k * LANES, LANES)]
                    bb = beta_v[pl.ds(k * LANES, LANES)]
                    xh = (xs[k] - mean_v) * scale
                    ost2[b, r, pl.ds(k * LANES, LANES)] = xh * g + bb

            fire_out(base, b)

            @pl.when(c + 2 < NCHUNK)
            def _():
                fire_gather(c + 2, b)

        fire_gather(0, 0)
        fire_gather(1, 1)

        def outer(g, _):
            process(2 * g, 0)
            process(2 * g + 1, 1)
            return 0

        lax.fori_loop(0, NCHUNK // 2, outer, 0)
        wait_out(wstart + (NCHUNK - 2) * CHUNK, 0)
        wait_out(wstart + (NCHUNK - 1) * CHUNK, 1)

    return sc_embed


_sc_embed = _make_sc_call()


@jax.jit
def _run(ids_flat, token_table, pos_table, gamma, beta):
    return _sc_embed(ids_flat, token_table, pos_table, gamma, beta)


def kernel(input_ids, token_table, pos_table, gamma, beta):
    ids3 = input_ids.reshape(NW, NCHUNK, CHUNK).astype(jnp.int32)
    out = _run(ids3, token_table, pos_table, gamma, beta)
    return out.reshape(BATCH, SEQ, EMBED)
